# Initial kernel scaffold; baseline (speedup 1.0000x reference)
import jax
import jax.numpy as jnp
from jax.experimental import pallas as pl


def kernel(x, row_embed_weight):
    raise NotImplementedError("write your pallas kernel here")


# speedup vs baseline: 1.1584x; 1.1584x over previous
import functools

import jax
import jax.numpy as jnp
from jax import lax
from jax.experimental import pallas as pl
from jax.experimental.pallas import tpu as pltpu
from jax.experimental.pallas import tpu_sc as plsc

_SEQ = 8192
_C = 13
_B = 4
_NUM_CORES = 2
_NUM_SUBCORES = 16
_NW = _NUM_CORES * _NUM_SUBCORES
_CHUNK = _SEQ // _NW
_L = 16


@functools.partial(
    pl.kernel,
    mesh=plsc.VectorSubcoreMesh(core_axis_name="c", subcore_axis_name="s"),
    out_type=jax.ShapeDtypeStruct((_B, _C, _SEQ), jnp.float32),
    compiler_params=pltpu.CompilerParams(needs_layout_passes=False),
    scratch_types=[
        pltpu.VMEM((_CHUNK * _C,), jnp.float32),
        pltpu.VMEM((_C, _CHUNK), jnp.float32),
    ],
)
def _pos_embed_sc(w_hbm, out_hbm, w_tile, out_tile):
    wid = lax.axis_index("s") * _NUM_CORES + lax.axis_index("c")
    base = wid * _CHUNK
    pltpu.sync_copy(w_hbm.at[pl.ds(base * _C, _CHUNK * _C)], w_tile)
    lane = lax.iota(jnp.int32, _L)
    for c in range(_C):
        for jb in range(0, _CHUNK, _L):
            idx = (lane + jb) * _C + c
            vals = plsc.load_gather(w_tile, [idx])
            out_tile[c, pl.ds(jb, _L)] = vals
    for b in range(_B):
        pltpu.sync_copy(out_tile, out_hbm.at[b, :, pl.ds(base, _CHUNK)])


def kernel(x, row_embed_weight):
    del x
    return _pos_embed_sc(row_embed_weight.reshape(-1))


# speedup vs baseline: 1.1608x; 1.0020x over previous
import functools

import jax
import jax.numpy as jnp
from jax import lax
from jax.experimental import pallas as pl
from jax.experimental.pallas import tpu as pltpu
from jax.experimental.pallas import tpu_sc as plsc

_SEQ = 8192
_C = 13
_B = 4
_NUM_CORES = 2
_NUM_SUBCORES = 16
_NW = _NUM_CORES * _NUM_SUBCORES
_CHUNK = _SEQ // _NW
_L = 16


@functools.partial(
    pl.kernel,
    mesh=plsc.VectorSubcoreMesh(core_axis_name="c", subcore_axis_name="s"),
    out_type=jax.ShapeDtypeStruct((_B, _C, _SEQ), jnp.float32),
    compiler_params=pltpu.CompilerParams(needs_layout_passes=False),
    scratch_types=[
        pltpu.VMEM((_CHUNK * _C,), jnp.float32),
        pltpu.VMEM((_C, _CHUNK), jnp.float32),
        pltpu.SemaphoreType.DMA,
    ],
)
def _pos_embed_sc(w_hbm, out_hbm, w_tile, out_tile, sem):
    wid = lax.axis_index("s") * _NUM_CORES + lax.axis_index("c")
    base = wid * _CHUNK
    pltpu.sync_copy(w_hbm.at[pl.ds(base * _C, _CHUNK * _C)], w_tile)
    lane = lax.iota(jnp.int32, _L)
    for c in range(_C):
        for jb in range(0, _CHUNK, _L):
            idx = (lane + jb) * _C + c
            vals = plsc.load_gather(w_tile, [idx])
            out_tile[c, pl.ds(jb, _L)] = vals
    copies = [
        pltpu.make_async_copy(out_tile, out_hbm.at[b, :, pl.ds(base, _CHUNK)], sem)
        for b in range(_B)
    ]
    for cp in copies:
        cp.start()
    for cp in copies:
        cp.wait()


def kernel(x, row_embed_weight):
    del x
    return _pos_embed_sc(row_embed_weight.reshape(-1))


# speedup vs baseline: 1.1936x; 1.0283x over previous
import functools

import jax
import jax.numpy as jnp
from jax import lax
from jax.experimental import pallas as pl
from jax.experimental.pallas import tpu as pltpu
from jax.experimental.pallas import tpu_sc as plsc

_SEQ = 8192
_C = 13
_B = 4
_NUM_CORES = 2
_NUM_SUBCORES = 16
_NW = _NUM_CORES * _NUM_SUBCORES
_CHUNK = _SEQ // _NW
_L = 16


@functools.partial(
    pl.kernel,
    mesh=plsc.VectorSubcoreMesh(core_axis_name="c", subcore_axis_name="s"),
    out_type=jax.ShapeDtypeStruct((_B, _C, _SEQ), jnp.float32),
    compiler_params=pltpu.CompilerParams(needs_layout_passes=False),
    scratch_types=[
        pltpu.VMEM((_CHUNK * _C,), jnp.float32),
        pltpu.VMEM((_C, _CHUNK), jnp.float32),
        pltpu.SemaphoreType.DMA,
    ],
)
def _pos_embed_sc(w_hbm, out_hbm, w_tile, out_tile, sem):
    wid = lax.axis_index("s") * _NUM_CORES + lax.axis_index("c")
    base = wid * _CHUNK
    pltpu.sync_copy(w_hbm.at[pl.ds(base * _C, _CHUNK * _C)], w_tile)
    lane13 = lax.iota(jnp.int32, _L) * _C
    copies = []
    for c0 in range(0, _C, 4):
        nc = min(4, _C - c0)
        for c in range(c0, c0 + nc):
            vals = [
                plsc.load_gather(w_tile, [lane13 + (jb * _C + c)])
                for jb in range(0, _CHUNK, _L)
            ]
            for i, jb in enumerate(range(0, _CHUNK, _L)):
                out_tile[c, pl.ds(jb, _L)] = vals[i]
        for b in range(_B):
            cp = pltpu.make_async_copy(
                out_tile.at[pl.ds(c0, nc), :],
                out_hbm.at[b, pl.ds(c0, nc), pl.ds(base, _CHUNK)],
                sem,
            )
            cp.start()
            copies.append(cp)
    for cp in copies:
        cp.wait()


def kernel(x, row_embed_weight):
    del x
    return _pos_embed_sc(row_embed_weight.reshape(-1))
